# baseline (speedup 1.0000x reference)
import jax
import jax.numpy as jnp
from jax.experimental import pallas as pl
from jax.experimental.pallas import tpu as pltpu

_TOP_K = 8
_N_EXPERTS = 64


def _gate_kernel(x_ref, w_ref, idx_ref, wgt_ref, log_ref, *, n_blocks):
    i = pl.program_id(0)

    @pl.when(i < n_blocks)
    def _():
        log_ref[i % 2] = jax.lax.dot_general(
            w_ref[...], x_ref[...],
            dimension_numbers=(((1,), (1,)), ((), ())),
            preferred_element_type=jnp.float32)

    logits = log_ref[(i + 1) % 2]
    row = jax.lax.broadcasted_iota(jnp.int32, logits.shape, 0)
    vals = logits
    sel_v = []
    sel_i = []
    for _ in range(_TOP_K):
        m = jnp.max(vals, axis=0, keepdims=True)
        cand = jnp.where(vals == m, row, _N_EXPERTS)
        a = jnp.min(cand, axis=0, keepdims=True)
        sel_v.append(m)
        sel_i.append(a)
        vals = jnp.where(row == a, -jnp.inf, vals)
    top_v = jnp.concatenate(sel_v, axis=0)
    top_i = jnp.concatenate(sel_i, axis=0)
    e = jnp.exp(top_v - top_v[0:1, :])
    wgt = e / jnp.sum(e, axis=0, keepdims=True)
    idx_ref[...] = top_i
    wgt_ref[...] = wgt


def kernel(hidden_states, weight):
    bsz, seq_len, dim = hidden_states.shape
    n_tokens = bsz * seq_len
    x = hidden_states.reshape(n_tokens, dim)
    block = 512
    n_blocks = n_tokens // block
    import functools
    body = functools.partial(_gate_kernel, n_blocks=n_blocks)
    idx_t, wgt_t = pl.pallas_call(
        body,
        grid=(n_blocks + 1,),
        compiler_params=pltpu.CompilerParams(
            dimension_semantics=("arbitrary",)),
        in_specs=[
            pl.BlockSpec((block, dim),
                         lambda i: (jnp.minimum(i, n_blocks - 1), 0)),
            pl.BlockSpec((_N_EXPERTS, dim), lambda i: (0, 0)),
        ],
        out_specs=[
            pl.BlockSpec((_TOP_K, block),
                         lambda i: (0, jnp.maximum(i - 1, 0))),
            pl.BlockSpec((_TOP_K, block),
                         lambda i: (0, jnp.maximum(i - 1, 0))),
        ],
        out_shape=[
            jax.ShapeDtypeStruct((_TOP_K, n_tokens), jnp.int32),
            jax.ShapeDtypeStruct((_TOP_K, n_tokens), jnp.float32),
        ],
        scratch_shapes=[pltpu.VMEM((2, _N_EXPERTS, block), jnp.float32)],
    )(x, weight)
    aux_loss = jnp.asarray(0.0, dtype=hidden_states.dtype)
    return idx_t.T, wgt_t.T.astype(hidden_states.dtype), aux_loss


# speedup vs baseline: 1.0433x; 1.0433x over previous
import jax
import jax.numpy as jnp
from jax.experimental import pallas as pl
from jax.experimental.pallas import tpu as pltpu

_TOP_K = 8
_N_EXPERTS = 64


def _matmul_into(x_ref, w_ref, dst_ref):
    dst_ref[...] = jax.lax.dot_general(
        w_ref[...], x_ref[...],
        dimension_numbers=(((1,), (1,)), ((), ())),
        preferred_element_type=jnp.float32)


def _epilogue_from(src_ref, idx_ref, wgt_ref):
    logits = src_ref[...]
    row = jax.lax.broadcasted_iota(jnp.int32, logits.shape, 0)
    vals = logits
    sel_v = []
    sel_i = []
    for _ in range(_TOP_K):
        m = jnp.max(vals, axis=0, keepdims=True)
        cand = jnp.where(vals == m, row, _N_EXPERTS)
        a = jnp.min(cand, axis=0, keepdims=True)
        sel_v.append(m)
        sel_i.append(a)
        vals = jnp.where(row == a, -jnp.inf, vals)
    top_v = jnp.concatenate(sel_v, axis=0)
    top_i = jnp.concatenate(sel_i, axis=0)
    e = jnp.exp(top_v - top_v[0:1, :])
    wgt = e / jnp.sum(e, axis=0, keepdims=True)
    idx_ref[...] = top_i
    wgt_ref[...] = wgt


def _gate_kernel(x_ref, w_ref, idx_ref, wgt_ref, la_ref, lb_ref, *, n_blocks):
    i = pl.program_id(0)
    even = jax.lax.rem(i, 2) == 0

    @pl.when(jnp.logical_and(even, i < n_blocks))
    def _():
        _matmul_into(x_ref, w_ref, la_ref)
        _epilogue_from(lb_ref, idx_ref, wgt_ref)

    @pl.when(jnp.logical_not(even))
    def _():
        _matmul_into(x_ref, w_ref, lb_ref)
        _epilogue_from(la_ref, idx_ref, wgt_ref)

    @pl.when(i >= n_blocks)
    def _():
        _epilogue_from(lb_ref, idx_ref, wgt_ref)


def kernel(hidden_states, weight):
    bsz, seq_len, dim = hidden_states.shape
    n_tokens = bsz * seq_len
    x = hidden_states.reshape(n_tokens, dim)
    block = 512
    n_blocks = n_tokens // block
    import functools
    body = functools.partial(_gate_kernel, n_blocks=n_blocks)
    idx_t, wgt_t = pl.pallas_call(
        body,
        grid=(n_blocks + 1,),
        compiler_params=pltpu.CompilerParams(
            dimension_semantics=("arbitrary",)),
        in_specs=[
            pl.BlockSpec((block, dim),
                         lambda i: (jnp.minimum(i, n_blocks - 1), 0)),
            pl.BlockSpec((_N_EXPERTS, dim), lambda i: (0, 0)),
        ],
        out_specs=[
            pl.BlockSpec((_TOP_K, block),
                         lambda i: (0, jnp.maximum(i - 1, 0))),
            pl.BlockSpec((_TOP_K, block),
                         lambda i: (0, jnp.maximum(i - 1, 0))),
        ],
        out_shape=[
            jax.ShapeDtypeStruct((_TOP_K, n_tokens), jnp.int32),
            jax.ShapeDtypeStruct((_TOP_K, n_tokens), jnp.float32),
        ],
        scratch_shapes=[pltpu.VMEM((_N_EXPERTS, block), jnp.float32),
                        pltpu.VMEM((_N_EXPERTS, block), jnp.float32)],
    )(x, weight)
    aux_loss = jnp.asarray(0.0, dtype=hidden_states.dtype)
    return idx_t.T, wgt_t.T.astype(hidden_states.dtype), aux_loss
